# baseline (speedup 1.0000x reference)
import functools

import jax
import jax.numpy as jnp
from jax import lax
from jax.experimental import pallas as pl
from jax.experimental.pallas import tpu as pltpu
from jax.experimental.pallas import tpu_sc as plsc

NC = 2
NS = 16
NW = NC * NS
CH = 80


def _sc_agg_body(src_hbm, dst_hbm, z_hbm, o0, o1,
                 src_w, da, db, r0, r1, agg_sh, si, sj, s0, s1, sadd, sz):
    n_nodes, width = agg_sh.shape
    _, n_ch, ch = src_hbm.shape
    rpt = n_nodes // NS

    c = lax.axis_index("c")
    s = lax.axis_index("s")
    wid = s * NC + c

    pltpu.async_copy(src_hbm.at[wid], src_w, sj)
    pltpu.async_copy(dst_hbm.at[wid, 0], da, si)

    def _zrow(i, _):
        def _zlane(j, _):
            r0[i, pl.ds(j * 16, 16)] = jnp.zeros((16,), jnp.float32)
            return 0
        return lax.fori_loop(0, width // 16, _zlane, 0)
    lax.fori_loop(0, ch, _zrow, 0)

    zfull = (-(-n_nodes // NS) + ch - 1) // ch
    z0 = s * zfull * ch
    nz = jnp.minimum(zfull, jnp.maximum(0, -(-(n_nodes - z0) // ch)))

    def _zc(i, _):
        pltpu.async_copy(r0, agg_sh.at[pl.ds(z0 + i * ch, ch)], sz)
        return 0
    lax.fori_loop(0, nz, _zc, 0)

    def _zw(i, _):
        pltpu.make_async_copy(r0, agg_sh.at[pl.ds(z0, ch)], sz).wait()
        return 0
    lax.fori_loop(0, nz, _zw, 0)

    pltpu.make_async_copy(src_hbm.at[wid], src_w, sj).wait()
    pltpu.async_copy(z_hbm.at[src_w.at[0]], r0, s0)
    plsc.subcore_barrier()

    def _half(t, rA, rB, dA, dB, gA, gB, first, last):
        pltpu.make_async_copy(z_hbm.at[src_w.at[0]], rA, gA).wait()
        if not first:
            pltpu.make_async_copy(rB, agg_sh.at[dB], sadd).wait()
        pltpu.make_async_copy(dst_hbm.at[wid, 0], dA, si).wait()
        if not last:
            pltpu.async_copy(dst_hbm.at[wid, t + 1], dB, si)
            pltpu.async_copy(z_hbm.at[src_w.at[t + 1]], rB, gB)
        pltpu.async_copy(rA, agg_sh.at[dA], sadd, add=True)

    def _pair(k, _):
        t = 2 * k

        @pl.when(k == 0)
        def _():
            _half(t, r0, r1, da, db, s0, s1, True, False)

        @pl.when(k > 0)
        def _():
            _half(t, r0, r1, da, db, s0, s1, False, False)
        _half(t + 1, r1, r0, db, da, s1, s0, False, False)
        return 0
    lax.fori_loop(0, (n_ch - 1) // 2, _pair, 0)
    _half(n_ch - 1, r0, r1, da, db, s0, s1, False, True)
    pltpu.make_async_copy(r0, agg_sh.at[da], sadd).wait()

    plsc.subcore_barrier()

    @pl.when(c == 0)
    def _():
        pltpu.sync_copy(agg_sh.at[pl.ds(s * rpt, rpt)], o0.at[pl.ds(s * rpt, rpt)])

    @pl.when(c == 1)
    def _():
        pltpu.sync_copy(agg_sh.at[pl.ds(s * rpt, rpt)], o1.at[pl.ds(s * rpt, rpt)])


@jax.jit
def _sc_agg(src, dst, z):
    n_nodes, width = z.shape
    mesh = plsc.VectorSubcoreMesh(core_axis_name="c", subcore_axis_name="s",
                                  num_cores=NC, num_subcores=NS)
    f = pl.kernel(
        _sc_agg_body,
        out_type=[jax.ShapeDtypeStruct((n_nodes, width), jnp.float32)] * 2,
        mesh=mesh,
        scratch_types=[
            pltpu.VMEM((src.shape[1], CH), jnp.int32),
            pltpu.VMEM((CH,), jnp.int32),
            pltpu.VMEM((CH,), jnp.int32),
            pltpu.VMEM((CH, width), jnp.float32),
            pltpu.VMEM((CH, width), jnp.float32),
            pltpu.VMEM_SHARED((n_nodes, width), jnp.float32),
            pltpu.SemaphoreType.DMA,
            pltpu.SemaphoreType.DMA,
            pltpu.SemaphoreType.DMA,
            pltpu.SemaphoreType.DMA,
            pltpu.SemaphoreType.DMA,
            pltpu.SemaphoreType.DMA,
        ],
        compiler_params=pltpu.CompilerParams(use_tc_tiling_on_sc=False),
    )
    return f(src, dst, z)


def _proj0_body(x_ref, w_ref, o_ref):
    bn = x_ref.shape[0]
    o_ref[:, :128] = jnp.dot(x_ref[...], w_ref[...],
                             preferred_element_type=jnp.float32)
    col = lax.broadcasted_iota(jnp.int32, (bn, 16), 1)
    o_ref[:, 128:144] = jnp.where(col == 0, 1.0, 0.0)


def _mid_body(a0_ref, a1_ref, z_ref, b_ref, w_ref, o_ref, inv_ref):
    a0 = a0_ref[...]
    a1 = a1_ref[...]
    z = z_ref[...]
    d = a0[:, 128:129] + a1[:, 128:129] + 1.0
    inv = 1.0 / d
    h = (a0[:, :128] + a1[:, :128] + z[:, :128]) * inv + b_ref[...]
    h = jnp.maximum(h, 0.0)
    o_ref[...] = jnp.dot(h, w_ref[...], preferred_element_type=jnp.float32)
    inv_ref[...] = inv


def _mid2_body(a0_ref, a1_ref, z_ref, inv_ref, b_ref, w_ref, o_ref):
    inv = inv_ref[...]
    h = (a0_ref[...] + a1_ref[...] + z_ref[...]) * inv + b_ref[...]
    h = jnp.maximum(h, 0.0)
    o_ref[...] = jnp.dot(h, w_ref[...], preferred_element_type=jnp.float32)


def _final_body(a0_ref, a1_ref, z_ref, inv_ref, b_ref, o_ref):
    bn, w = z_ref.shape
    t = (a0_ref[...] + a1_ref[...] + z_ref[...]) * inv_ref[...] + b_ref[...]
    col = lax.broadcasted_iota(jnp.int32, (bn, w), 1)
    valid = col < 47
    tm = jnp.where(valid, t, -1e30)
    m = jnp.max(tm, axis=1, keepdims=True)
    e = jnp.exp(tm - m)
    ssum = jnp.sum(e, axis=1, keepdims=True)
    o_ref[...] = jnp.where(valid, t - m - jnp.log(ssum), 0.0)


def _rows_spec(bn, w):
    return pl.BlockSpec((bn, w), lambda i: (i, 0))


def _full_spec(shape):
    return pl.BlockSpec(shape, lambda i: tuple(0 for _ in shape))


def kernel(x, edge_index, W0, b0, W1, b1, W2, b2):
    n, d = x.shape
    dh = W1.shape[0]
    ncls = W2.shape[1]
    e = edge_index.shape[1]
    epw = e // NW
    src = edge_index[0].astype(jnp.int32).reshape(NW, epw // CH, CH)
    dst = edge_index[1].astype(jnp.int32).reshape(NW, epw // CH, CH)

    bn = 1000
    grid = (n // bn,)

    z1 = pl.pallas_call(
        _proj0_body,
        grid=grid,
        in_specs=[_rows_spec(bn, d), _full_spec((d, dh))],
        out_specs=_rows_spec(bn, dh + 16),
        out_shape=jax.ShapeDtypeStruct((n, dh + 16), jnp.float32),
    )(x, W0)

    a0, a1 = _sc_agg(src, dst, z1)

    z2, inv = pl.pallas_call(
        _mid_body,
        grid=grid,
        in_specs=[_rows_spec(bn, dh + 16), _rows_spec(bn, dh + 16),
                  _rows_spec(bn, dh + 16), _full_spec((1, dh)),
                  _full_spec((dh, dh))],
        out_specs=[_rows_spec(bn, dh), pl.BlockSpec((bn, 1), lambda i: (i, 0))],
        out_shape=[jax.ShapeDtypeStruct((n, dh), jnp.float32),
                   jax.ShapeDtypeStruct((n, 1), jnp.float32)],
    )(a0, a1, z1, b0.reshape(1, dh), W1)

    a0, a1 = _sc_agg(src, dst, z2)

    ncp = 48
    W2p = jnp.pad(W2, ((0, 0), (0, ncp - ncls)))
    b2p = jnp.pad(b2, (0, ncp - ncls)).reshape(1, ncp)
    z3 = pl.pallas_call(
        _mid2_body,
        grid=grid,
        in_specs=[_rows_spec(bn, dh), _rows_spec(bn, dh), _rows_spec(bn, dh),
                  pl.BlockSpec((bn, 1), lambda i: (i, 0)),
                  _full_spec((1, dh)), _full_spec((dh, ncp))],
        out_specs=_rows_spec(bn, ncp),
        out_shape=jax.ShapeDtypeStruct((n, ncp), jnp.float32),
    )(a0, a1, z2, inv, b1.reshape(1, dh), W2p)

    a0, a1 = _sc_agg(src, dst, z3)

    out = pl.pallas_call(
        _final_body,
        grid=grid,
        in_specs=[_rows_spec(bn, ncp), _rows_spec(bn, ncp), _rows_spec(bn, ncp),
                  pl.BlockSpec((bn, 1), lambda i: (i, 0)),
                  _full_spec((1, ncp))],
        out_specs=_rows_spec(bn, ncp),
        out_shape=jax.ShapeDtypeStruct((n, ncp), jnp.float32),
    )(a0, a1, z3, inv, b2p)

    return out[:, :ncls]


# speedup vs baseline: 1.2462x; 1.2462x over previous
import functools

import jax
import jax.numpy as jnp
from jax import lax
from jax.experimental import pallas as pl
from jax.experimental.pallas import tpu as pltpu
from jax.experimental.pallas import tpu_sc as plsc

NC = 2
NS = 16
NW = NC * NS
CH = 80


def _sc_agg_body(src_hbm, dst_hbm, z_hbm, o0, o1,
                 src_w, da, db, r0, r1, agg_sh, si, sj, s0, s1, sadd, sz):
    n_nodes, width = agg_sh.shape
    _, n_ch, ch = src_hbm.shape
    rpt = n_nodes // NS

    c = lax.axis_index("c")
    s = lax.axis_index("s")
    wid = s * NC + c

    pltpu.async_copy(src_hbm.at[wid], src_w, sj)
    pltpu.async_copy(dst_hbm.at[wid, 0], da, si)

    def _zrow(i, _):
        def _zlane(j, _):
            r0[i, pl.ds(j * 16, 16)] = jnp.zeros((16,), jnp.float32)
            return 0
        return lax.fori_loop(0, width // 16, _zlane, 0)
    lax.fori_loop(0, ch, _zrow, 0)

    zfull = (-(-n_nodes // NS) + ch - 1) // ch
    z0 = s * zfull * ch
    nz = jnp.minimum(zfull, jnp.maximum(0, -(-(n_nodes - z0) // ch)))

    def _zc(i, _):
        pltpu.async_copy(r0, agg_sh.at[pl.ds(z0 + i * ch, ch)], sz)
        return 0
    lax.fori_loop(0, nz, _zc, 0)

    def _zw(i, _):
        pltpu.make_async_copy(r0, agg_sh.at[pl.ds(z0, ch)], sz).wait()
        return 0
    lax.fori_loop(0, nz, _zw, 0)

    pltpu.async_copy(dst_hbm.at[wid, 1], db, si)
    pltpu.make_async_copy(src_hbm.at[wid], src_w, sj).wait()
    pltpu.async_copy(z_hbm.at[src_w.at[0]], r0, s0)
    plsc.subcore_barrier()

    def _pair(k, _):
        a = 2 * k
        pltpu.async_copy(z_hbm.at[src_w.at[a + 1]], r1, s1)
        pltpu.make_async_copy(dst_hbm.at[wid, 0], da, si).wait()
        pltpu.make_async_copy(dst_hbm.at[wid, 0], db, si).wait()
        pltpu.make_async_copy(z_hbm.at[src_w.at[0]], r0, s0).wait()
        pltpu.sync_copy(r0, agg_sh.at[da], add=True)
        pltpu.async_copy(dst_hbm.at[wid, a + 2], da, si)
        pltpu.async_copy(z_hbm.at[src_w.at[a + 2]], r0, s0)
        pltpu.make_async_copy(z_hbm.at[src_w.at[0]], r1, s1).wait()
        pltpu.sync_copy(r1, agg_sh.at[db], add=True)

        @pl.when(a + 3 < n_ch)
        def _():
            pltpu.async_copy(dst_hbm.at[wid, a + 3], db, si)
        return 0
    lax.fori_loop(0, (n_ch - 1) // 2, _pair, 0)
    pltpu.make_async_copy(dst_hbm.at[wid, 0], da, si).wait()
    pltpu.make_async_copy(z_hbm.at[src_w.at[0]], r0, s0).wait()
    pltpu.sync_copy(r0, agg_sh.at[da], add=True)

    plsc.subcore_barrier()

    @pl.when(c == 0)
    def _():
        pltpu.sync_copy(agg_sh.at[pl.ds(s * rpt, rpt)], o0.at[pl.ds(s * rpt, rpt)])

    @pl.when(c == 1)
    def _():
        pltpu.sync_copy(agg_sh.at[pl.ds(s * rpt, rpt)], o1.at[pl.ds(s * rpt, rpt)])


@jax.jit
def _sc_agg(src, dst, z):
    n_nodes, width = z.shape
    mesh = plsc.VectorSubcoreMesh(core_axis_name="c", subcore_axis_name="s",
                                  num_cores=NC, num_subcores=NS)
    f = pl.kernel(
        _sc_agg_body,
        out_type=[jax.ShapeDtypeStruct((n_nodes, width), jnp.float32)] * 2,
        mesh=mesh,
        scratch_types=[
            pltpu.VMEM((src.shape[1], CH), jnp.int32),
            pltpu.VMEM((CH,), jnp.int32),
            pltpu.VMEM((CH,), jnp.int32),
            pltpu.VMEM((CH, width), jnp.float32),
            pltpu.VMEM((CH, width), jnp.float32),
            pltpu.VMEM_SHARED((n_nodes, width), jnp.float32),
            pltpu.SemaphoreType.DMA,
            pltpu.SemaphoreType.DMA,
            pltpu.SemaphoreType.DMA,
            pltpu.SemaphoreType.DMA,
            pltpu.SemaphoreType.DMA,
            pltpu.SemaphoreType.DMA,
        ],
        compiler_params=pltpu.CompilerParams(use_tc_tiling_on_sc=False),
    )
    return f(src, dst, z)


def _proj0_body(x_ref, w_ref, o_ref):
    bn = x_ref.shape[0]
    o_ref[:, :128] = jnp.dot(x_ref[...], w_ref[...],
                             preferred_element_type=jnp.float32)
    col = lax.broadcasted_iota(jnp.int32, (bn, 16), 1)
    o_ref[:, 128:144] = jnp.where(col == 0, 1.0, 0.0)


def _mid_body(a0_ref, a1_ref, z_ref, b_ref, w_ref, o_ref, inv_ref):
    a0 = a0_ref[...]
    a1 = a1_ref[...]
    z = z_ref[...]
    d = a0[:, 128:129] + a1[:, 128:129] + 1.0
    inv = 1.0 / d
    h = (a0[:, :128] + a1[:, :128] + z[:, :128]) * inv + b_ref[...]
    h = jnp.maximum(h, 0.0)
    o_ref[...] = jnp.dot(h, w_ref[...], preferred_element_type=jnp.float32)
    inv_ref[...] = inv


def _mid2_body(a0_ref, a1_ref, z_ref, inv_ref, b_ref, w_ref, o_ref):
    inv = inv_ref[...]
    h = (a0_ref[...] + a1_ref[...] + z_ref[...]) * inv + b_ref[...]
    h = jnp.maximum(h, 0.0)
    o_ref[...] = jnp.dot(h, w_ref[...], preferred_element_type=jnp.float32)


def _final_body(a0_ref, a1_ref, z_ref, inv_ref, b_ref, o_ref):
    bn, w = z_ref.shape
    t = (a0_ref[...] + a1_ref[...] + z_ref[...]) * inv_ref[...] + b_ref[...]
    col = lax.broadcasted_iota(jnp.int32, (bn, w), 1)
    valid = col < 47
    tm = jnp.where(valid, t, -1e30)
    m = jnp.max(tm, axis=1, keepdims=True)
    e = jnp.exp(tm - m)
    ssum = jnp.sum(e, axis=1, keepdims=True)
    o_ref[...] = jnp.where(valid, t - m - jnp.log(ssum), 0.0)


def _rows_spec(bn, w):
    return pl.BlockSpec((bn, w), lambda i: (i, 0))


def _full_spec(shape):
    return pl.BlockSpec(shape, lambda i: tuple(0 for _ in shape))


def kernel(x, edge_index, W0, b0, W1, b1, W2, b2):
    n, d = x.shape
    dh = W1.shape[0]
    ncls = W2.shape[1]
    e = edge_index.shape[1]
    epw = e // NW
    src = edge_index[0].astype(jnp.int32).reshape(NW, epw // CH, CH)
    dst = edge_index[1].astype(jnp.int32).reshape(NW, epw // CH, CH)

    bn = 1000
    grid = (n // bn,)

    z1 = pl.pallas_call(
        _proj0_body,
        grid=grid,
        in_specs=[_rows_spec(bn, d), _full_spec((d, dh))],
        out_specs=_rows_spec(bn, dh + 16),
        out_shape=jax.ShapeDtypeStruct((n, dh + 16), jnp.float32),
    )(x, W0)

    a0, a1 = _sc_agg(src, dst, z1)

    z2, inv = pl.pallas_call(
        _mid_body,
        grid=grid,
        in_specs=[_rows_spec(bn, dh + 16), _rows_spec(bn, dh + 16),
                  _rows_spec(bn, dh + 16), _full_spec((1, dh)),
                  _full_spec((dh, dh))],
        out_specs=[_rows_spec(bn, dh), pl.BlockSpec((bn, 1), lambda i: (i, 0))],
        out_shape=[jax.ShapeDtypeStruct((n, dh), jnp.float32),
                   jax.ShapeDtypeStruct((n, 1), jnp.float32)],
    )(a0, a1, z1, b0.reshape(1, dh), W1)

    a0, a1 = _sc_agg(src, dst, z2)

    ncp = 48
    W2p = jnp.pad(W2, ((0, 0), (0, ncp - ncls)))
    b2p = jnp.pad(b2, (0, ncp - ncls)).reshape(1, ncp)
    z3 = pl.pallas_call(
        _mid2_body,
        grid=grid,
        in_specs=[_rows_spec(bn, dh), _rows_spec(bn, dh), _rows_spec(bn, dh),
                  pl.BlockSpec((bn, 1), lambda i: (i, 0)),
                  _full_spec((1, dh)), _full_spec((dh, ncp))],
        out_specs=_rows_spec(bn, ncp),
        out_shape=jax.ShapeDtypeStruct((n, ncp), jnp.float32),
    )(a0, a1, z2, inv, b1.reshape(1, dh), W2p)

    a0, a1 = _sc_agg(src, dst, z3)

    out = pl.pallas_call(
        _final_body,
        grid=grid,
        in_specs=[_rows_spec(bn, ncp), _rows_spec(bn, ncp), _rows_spec(bn, ncp),
                  pl.BlockSpec((bn, 1), lambda i: (i, 0)),
                  _full_spec((1, ncp))],
        out_specs=_rows_spec(bn, ncp),
        out_shape=jax.ShapeDtypeStruct((n, ncp), jnp.float32),
    )(a0, a1, z3, inv, b2p)

    return out[:, :ncls]


# speedup vs baseline: 1.2693x; 1.0185x over previous
import functools

import jax
import jax.numpy as jnp
from jax import lax
from jax.experimental import pallas as pl
from jax.experimental.pallas import tpu as pltpu
from jax.experimental.pallas import tpu_sc as plsc

NC = 2
NS = 16
NW = NC * NS
CH = 80


def _sc_agg_body(src_hbm, dst_hbm, z_hbm, o0, o1,
                 src_w, da, db, r0, r1, agg_sh, si, sj, s0, s1, sadd, sz):
    n_nodes, width = agg_sh.shape
    _, n_ch, ch = src_hbm.shape
    rpt = n_nodes // NS

    c = lax.axis_index("c")
    s = lax.axis_index("s")
    wid = s * NC + c

    pltpu.async_copy(src_hbm.at[wid], src_w, sj)
    pltpu.async_copy(dst_hbm.at[wid, 0], da, si)

    def _zrow(i, _):
        def _zlane(j, _):
            r0[i, pl.ds(j * 16, 16)] = jnp.zeros((16,), jnp.float32)
            return 0
        return lax.fori_loop(0, width // 16, _zlane, 0)
    lax.fori_loop(0, ch, _zrow, 0)

    zfull = (-(-n_nodes // NS) + ch - 1) // ch
    z0 = s * zfull * ch
    nz = jnp.minimum(zfull, jnp.maximum(0, -(-(n_nodes - z0) // ch)))

    def _zc(i, _):
        pltpu.async_copy(r0, agg_sh.at[pl.ds(z0 + i * ch, ch)], sz)
        return 0
    lax.fori_loop(0, nz, _zc, 0)

    def _zw(i, _):
        pltpu.make_async_copy(r0, agg_sh.at[pl.ds(z0, ch)], sz).wait()
        return 0
    lax.fori_loop(0, nz, _zw, 0)

    pltpu.async_copy(dst_hbm.at[wid, 1], db, si)
    pltpu.make_async_copy(src_hbm.at[wid], src_w, sj).wait()
    pltpu.async_copy(z_hbm.at[src_w.at[0]], r0, s0)
    plsc.subcore_barrier()

    def _pair(k, _):
        a = 2 * k
        pltpu.async_copy(z_hbm.at[src_w.at[a + 1]], r1, s1)
        pltpu.make_async_copy(dst_hbm.at[wid, 0], da, si).wait()
        pltpu.make_async_copy(dst_hbm.at[wid, 0], db, si).wait()
        pltpu.make_async_copy(z_hbm.at[src_w.at[0]], r0, s0).wait()
        pltpu.sync_copy(r0, agg_sh.at[da], add=True)
        pltpu.async_copy(dst_hbm.at[wid, a + 2], da, si)
        pltpu.async_copy(z_hbm.at[src_w.at[a + 2]], r0, s0)
        pltpu.make_async_copy(z_hbm.at[src_w.at[0]], r1, s1).wait()
        pltpu.sync_copy(r1, agg_sh.at[db], add=True)

        @pl.when(a + 3 < n_ch)
        def _():
            pltpu.async_copy(dst_hbm.at[wid, a + 3], db, si)
        return 0
    lax.fori_loop(0, (n_ch - 1) // 2, _pair, 0)
    pltpu.make_async_copy(dst_hbm.at[wid, 0], da, si).wait()
    pltpu.make_async_copy(z_hbm.at[src_w.at[0]], r0, s0).wait()
    pltpu.sync_copy(r0, agg_sh.at[da], add=True)

    plsc.subcore_barrier()

    @pl.when(c == 0)
    def _():
        pltpu.sync_copy(agg_sh.at[pl.ds(s * rpt, rpt)], o0.at[pl.ds(s * rpt, rpt)])

    @pl.when(c == 1)
    def _():
        pltpu.sync_copy(agg_sh.at[pl.ds(s * rpt, rpt)], o1.at[pl.ds(s * rpt, rpt)])


@jax.jit
def _sc_agg(src, dst, z):
    n_nodes, width = z.shape
    mesh = plsc.VectorSubcoreMesh(core_axis_name="c", subcore_axis_name="s",
                                  num_cores=NC, num_subcores=NS)
    f = pl.kernel(
        _sc_agg_body,
        out_type=[jax.ShapeDtypeStruct((n_nodes, width), jnp.float32)] * 2,
        mesh=mesh,
        scratch_types=[
            pltpu.VMEM((src.shape[1], CH), jnp.int32),
            pltpu.VMEM((CH,), jnp.int32),
            pltpu.VMEM((CH,), jnp.int32),
            pltpu.VMEM((CH, width), jnp.float32),
            pltpu.VMEM((CH, width), jnp.float32),
            pltpu.VMEM_SHARED((n_nodes, width), jnp.float32),
            pltpu.SemaphoreType.DMA,
            pltpu.SemaphoreType.DMA,
            pltpu.SemaphoreType.DMA,
            pltpu.SemaphoreType.DMA,
            pltpu.SemaphoreType.DMA,
            pltpu.SemaphoreType.DMA,
        ],
        compiler_params=pltpu.CompilerParams(use_tc_tiling_on_sc=False),
    )
    return f(src, dst, z)


def _proj0_body(x_ref, w_ref, o_ref):
    bn = x_ref.shape[0]
    o_ref[:, :128] = jnp.dot(x_ref[...], w_ref[...],
                             preferred_element_type=jnp.float32)
    col = lax.broadcasted_iota(jnp.int32, (bn, 16), 1)
    o_ref[:, 128:144] = jnp.where(col == 0, 1.0, 0.0)


def _mid_body(a0_ref, a1_ref, z_ref, b_ref, w_ref, o_ref, inv_ref):
    a0 = a0_ref[...]
    a1 = a1_ref[...]
    z = z_ref[...]
    d = a0[:, 128:129] + a1[:, 128:129] + 1.0
    inv = 1.0 / d
    h = (a0[:, :128] + a1[:, :128] + z[:, :128]) * inv + b_ref[...]
    h = jnp.maximum(h, 0.0)
    o_ref[...] = jnp.dot(h, w_ref[...], preferred_element_type=jnp.float32)
    inv_ref[...] = inv


def _mid2_body(a0_ref, a1_ref, z_ref, inv_ref, b_ref, w_ref, o_ref):
    inv = inv_ref[...]
    h = (a0_ref[...] + a1_ref[...] + z_ref[...]) * inv + b_ref[...]
    h = jnp.maximum(h, 0.0)
    o_ref[...] = jnp.dot(h, w_ref[...], preferred_element_type=jnp.float32)


def _final_body(a0_ref, a1_ref, z_ref, inv_ref, b_ref, o_ref):
    bn, w = z_ref.shape
    ncls = o_ref.shape[1]
    t = (a0_ref[...] + a1_ref[...] + z_ref[...]) * inv_ref[...] + b_ref[...]
    col = lax.broadcasted_iota(jnp.int32, (bn, w), 1)
    valid = col < ncls
    tm = jnp.where(valid, t, -1e30)
    m = jnp.max(tm, axis=1, keepdims=True)
    e = jnp.exp(tm - m)
    ssum = jnp.sum(e, axis=1, keepdims=True)
    o_ref[...] = (t - m - jnp.log(ssum))[:, :ncls]


def _rows_spec(bn, w):
    return pl.BlockSpec((bn, w), lambda i: (i, 0))


def _full_spec(shape):
    return pl.BlockSpec(shape, lambda i: tuple(0 for _ in shape))


def kernel(x, edge_index, W0, b0, W1, b1, W2, b2):
    n, d = x.shape
    dh = W1.shape[0]
    ncls = W2.shape[1]
    e = edge_index.shape[1]
    epw = e // NW
    src = edge_index[0].astype(jnp.int32).reshape(NW, epw // CH, CH)
    dst = edge_index[1].astype(jnp.int32).reshape(NW, epw // CH, CH)

    bn = 2000
    grid = (n // bn,)

    z1 = pl.pallas_call(
        _proj0_body,
        grid=grid,
        in_specs=[_rows_spec(bn, d), _full_spec((d, dh))],
        out_specs=_rows_spec(bn, dh + 16),
        out_shape=jax.ShapeDtypeStruct((n, dh + 16), jnp.float32),
    )(x, W0)

    a0, a1 = _sc_agg(src, dst, z1)

    z2, inv = pl.pallas_call(
        _mid_body,
        grid=grid,
        in_specs=[_rows_spec(bn, dh + 16), _rows_spec(bn, dh + 16),
                  _rows_spec(bn, dh + 16), _full_spec((1, dh)),
                  _full_spec((dh, dh))],
        out_specs=[_rows_spec(bn, dh), pl.BlockSpec((bn, 1), lambda i: (i, 0))],
        out_shape=[jax.ShapeDtypeStruct((n, dh), jnp.float32),
                   jax.ShapeDtypeStruct((n, 1), jnp.float32)],
    )(a0, a1, z1, b0.reshape(1, dh), W1)

    a0, a1 = _sc_agg(src, dst, z2)

    ncp = 48
    W2p = jnp.pad(W2, ((0, 0), (0, ncp - ncls)))
    b2p = jnp.pad(b2, (0, ncp - ncls)).reshape(1, ncp)
    z3 = pl.pallas_call(
        _mid2_body,
        grid=grid,
        in_specs=[_rows_spec(bn, dh), _rows_spec(bn, dh), _rows_spec(bn, dh),
                  pl.BlockSpec((bn, 1), lambda i: (i, 0)),
                  _full_spec((1, dh)), _full_spec((dh, ncp))],
        out_specs=_rows_spec(bn, ncp),
        out_shape=jax.ShapeDtypeStruct((n, ncp), jnp.float32),
    )(a0, a1, z2, inv, b1.reshape(1, dh), W2p)

    a0, a1 = _sc_agg(src, dst, z3)

    out = pl.pallas_call(
        _final_body,
        grid=grid,
        in_specs=[_rows_spec(bn, ncp), _rows_spec(bn, ncp), _rows_spec(bn, ncp),
                  pl.BlockSpec((bn, 1), lambda i: (i, 0)),
                  _full_spec((1, ncp))],
        out_specs=_rows_spec(bn, ncls),
        out_shape=jax.ShapeDtypeStruct((n, ncls), jnp.float32),
    )(a0, a1, z3, inv, b2p)

    return out


# speedup vs baseline: 1.2992x; 1.0236x over previous
import functools

import jax
import jax.numpy as jnp
from jax import lax
from jax.experimental import pallas as pl
from jax.experimental.pallas import tpu as pltpu
from jax.experimental.pallas import tpu_sc as plsc

NC = 2
NS = 16
NW = NC * NS
CH = 80


def _sc_agg_body(ei_hbm, z_hbm, o0, o1,
                 src_w, da, db, r0, r1, agg_sh, si, sj, s0, s1, sadd, sz):
    n_nodes, width = agg_sh.shape
    _, _, n_ch, ch = ei_hbm.shape
    rpt = n_nodes // NS

    c = lax.axis_index("c")
    s = lax.axis_index("s")
    wid = s * NC + c

    pltpu.async_copy(ei_hbm.at[0, wid], src_w, sj)
    pltpu.async_copy(ei_hbm.at[1, wid, 0], da, si)

    def _zrow(i, _):
        def _zlane(j, _):
            r0[i, pl.ds(j * 16, 16)] = jnp.zeros((16,), jnp.float32)
            return 0
        return lax.fori_loop(0, width // 16, _zlane, 0)
    lax.fori_loop(0, ch, _zrow, 0)

    zfull = (-(-n_nodes // NS) + ch - 1) // ch
    z0 = s * zfull * ch
    nz = jnp.minimum(zfull, jnp.maximum(0, -(-(n_nodes - z0) // ch)))

    def _zc(i, _):
        pltpu.async_copy(r0, agg_sh.at[pl.ds(z0 + i * ch, ch)], sz)
        return 0
    lax.fori_loop(0, nz, _zc, 0)

    def _zw(i, _):
        pltpu.make_async_copy(r0, agg_sh.at[pl.ds(z0, ch)], sz).wait()
        return 0
    lax.fori_loop(0, nz, _zw, 0)

    pltpu.async_copy(ei_hbm.at[1, wid, 1], db, si)
    pltpu.make_async_copy(ei_hbm.at[0, wid], src_w, sj).wait()
    pltpu.async_copy(z_hbm.at[src_w.at[0]], r0, s0)
    plsc.subcore_barrier()

    def _pair(k, _):
        a = 2 * k
        pltpu.async_copy(z_hbm.at[src_w.at[a + 1]], r1, s1)
        pltpu.make_async_copy(ei_hbm.at[1, wid, 0], da, si).wait()
        pltpu.make_async_copy(ei_hbm.at[1, wid, 0], db, si).wait()
        pltpu.make_async_copy(z_hbm.at[src_w.at[0]], r0, s0).wait()
        pltpu.sync_copy(r0, agg_sh.at[da], add=True)
        pltpu.async_copy(ei_hbm.at[1, wid, a + 2], da, si)
        pltpu.async_copy(z_hbm.at[src_w.at[a + 2]], r0, s0)
        pltpu.make_async_copy(z_hbm.at[src_w.at[0]], r1, s1).wait()
        pltpu.sync_copy(r1, agg_sh.at[db], add=True)

        @pl.when(a + 3 < n_ch)
        def _():
            pltpu.async_copy(ei_hbm.at[1, wid, a + 3], db, si)
        return 0
    lax.fori_loop(0, (n_ch - 1) // 2, _pair, 0)
    pltpu.make_async_copy(ei_hbm.at[1, wid, 0], da, si).wait()
    pltpu.make_async_copy(z_hbm.at[src_w.at[0]], r0, s0).wait()
    pltpu.sync_copy(r0, agg_sh.at[da], add=True)

    plsc.subcore_barrier()

    @pl.when(c == 0)
    def _():
        pltpu.sync_copy(agg_sh.at[pl.ds(s * rpt, rpt)], o0.at[pl.ds(s * rpt, rpt)])

    @pl.when(c == 1)
    def _():
        pltpu.sync_copy(agg_sh.at[pl.ds(s * rpt, rpt)], o1.at[pl.ds(s * rpt, rpt)])


@jax.jit
def _sc_agg(ei, z):
    n_nodes, width = z.shape
    mesh = plsc.VectorSubcoreMesh(core_axis_name="c", subcore_axis_name="s",
                                  num_cores=NC, num_subcores=NS)
    f = pl.kernel(
        _sc_agg_body,
        out_type=[jax.ShapeDtypeStruct((n_nodes, width), jnp.float32)] * 2,
        mesh=mesh,
        scratch_types=[
            pltpu.VMEM((ei.shape[2], CH), jnp.int32),
            pltpu.VMEM((CH,), jnp.int32),
            pltpu.VMEM((CH,), jnp.int32),
            pltpu.VMEM((CH, width), jnp.float32),
            pltpu.VMEM((CH, width), jnp.float32),
            pltpu.VMEM_SHARED((n_nodes, width), jnp.float32),
            pltpu.SemaphoreType.DMA,
            pltpu.SemaphoreType.DMA,
            pltpu.SemaphoreType.DMA,
            pltpu.SemaphoreType.DMA,
            pltpu.SemaphoreType.DMA,
            pltpu.SemaphoreType.DMA,
        ],
        compiler_params=pltpu.CompilerParams(use_tc_tiling_on_sc=False),
    )
    return f(ei, z)


def _proj0_body(x_ref, w_ref, o_ref):
    bn = x_ref.shape[0]
    o_ref[:, :128] = jnp.dot(x_ref[...], w_ref[...],
                             preferred_element_type=jnp.float32)
    col = lax.broadcasted_iota(jnp.int32, (bn, 16), 1)
    o_ref[:, 128:144] = jnp.where(col == 0, 1.0, 0.0)


def _mid_body(a0_ref, a1_ref, z_ref, b_ref, w_ref, o_ref, inv_ref):
    a0 = a0_ref[...]
    a1 = a1_ref[...]
    z = z_ref[...]
    d = a0[:, 128:129] + a1[:, 128:129] + 1.0
    inv = 1.0 / d
    h = (a0[:, :128] + a1[:, :128] + z[:, :128]) * inv + b_ref[...][None, :]
    h = jnp.maximum(h, 0.0)
    o_ref[...] = jnp.dot(h, w_ref[...], preferred_element_type=jnp.float32)
    inv_ref[...] = inv


def _mid2_body(a0_ref, a1_ref, z_ref, inv_ref, b_ref, w_ref, o_ref):
    bn = z_ref.shape[0]
    ncp = o_ref.shape[1]
    ncls = w_ref.shape[1]
    inv = inv_ref[...]
    h = (a0_ref[...] + a1_ref[...] + z_ref[...]) * inv + b_ref[...][None, :]
    h = jnp.maximum(h, 0.0)
    mm = jnp.dot(h, w_ref[...], preferred_element_type=jnp.float32)
    o_ref[...] = jnp.concatenate(
        [mm, jnp.zeros((bn, ncp - ncls), jnp.float32)], axis=1)


def _final_body(a0_ref, a1_ref, z_ref, inv_ref, b_ref, o_ref):
    bn, w = z_ref.shape
    ncls = o_ref.shape[1]
    bp = jnp.concatenate(
        [b_ref[...], jnp.zeros((w - ncls,), jnp.float32)])[None, :]
    t = (a0_ref[...] + a1_ref[...] + z_ref[...]) * inv_ref[...] + bp
    col = lax.broadcasted_iota(jnp.int32, (bn, w), 1)
    valid = col < ncls
    tm = jnp.where(valid, t, -1e30)
    m = jnp.max(tm, axis=1, keepdims=True)
    e = jnp.exp(tm - m)
    ssum = jnp.sum(e, axis=1, keepdims=True)
    o_ref[...] = (t - m - jnp.log(ssum))[:, :ncls]


def _rows_spec(bn, w):
    return pl.BlockSpec((bn, w), lambda i: (i, 0))


def _full_spec(shape):
    return pl.BlockSpec(shape, lambda i: tuple(0 for _ in shape))


def kernel(x, edge_index, W0, b0, W1, b1, W2, b2):
    n, d = x.shape
    dh = W1.shape[0]
    ncls = W2.shape[1]
    e = edge_index.shape[1]
    epw = e // NW
    ei = edge_index.astype(jnp.int32).reshape(2, NW, epw // CH, CH)

    bn = 2000
    grid = (n // bn,)

    z1 = pl.pallas_call(
        _proj0_body,
        grid=grid,
        in_specs=[_rows_spec(bn, d), _full_spec((d, dh))],
        out_specs=_rows_spec(bn, dh + 16),
        out_shape=jax.ShapeDtypeStruct((n, dh + 16), jnp.float32),
    )(x, W0)

    a0, a1 = _sc_agg(ei, z1)

    z2, inv = pl.pallas_call(
        _mid_body,
        grid=grid,
        in_specs=[_rows_spec(bn, dh + 16), _rows_spec(bn, dh + 16),
                  _rows_spec(bn, dh + 16), _full_spec((dh,)),
                  _full_spec((dh, dh))],
        out_specs=[_rows_spec(bn, dh), pl.BlockSpec((bn, 1), lambda i: (i, 0))],
        out_shape=[jax.ShapeDtypeStruct((n, dh), jnp.float32),
                   jax.ShapeDtypeStruct((n, 1), jnp.float32)],
    )(a0, a1, z1, b0, W1)

    a0, a1 = _sc_agg(ei, z2)

    ncp = 48
    z3 = pl.pallas_call(
        _mid2_body,
        grid=grid,
        in_specs=[_rows_spec(bn, dh), _rows_spec(bn, dh), _rows_spec(bn, dh),
                  pl.BlockSpec((bn, 1), lambda i: (i, 0)),
                  _full_spec((dh,)), _full_spec((dh, ncls))],
        out_specs=_rows_spec(bn, ncp),
        out_shape=jax.ShapeDtypeStruct((n, ncp), jnp.float32),
    )(a0, a1, z2, inv, b1, W2)

    a0, a1 = _sc_agg(ei, z3)

    out = pl.pallas_call(
        _final_body,
        grid=grid,
        in_specs=[_rows_spec(bn, ncp), _rows_spec(bn, ncp), _rows_spec(bn, ncp),
                  pl.BlockSpec((bn, 1), lambda i: (i, 0)),
                  _full_spec((ncls,))],
        out_specs=_rows_spec(bn, ncls),
        out_shape=jax.ShapeDtypeStruct((n, ncls), jnp.float32),
    )(a0, a1, z3, inv, b2)

    return out


# speedup vs baseline: 1.3069x; 1.0059x over previous
import functools

import jax
import jax.numpy as jnp
from jax import lax
from jax.experimental import pallas as pl
from jax.experimental.pallas import tpu as pltpu
from jax.experimental.pallas import tpu_sc as plsc

NC = 2
NS = 16
NW = NC * NS
CH = 80


def _sc_agg_body(ei_hbm, z_hbm, o0, o1,
                 src_w, da, db, r0, r1, agg_sh, si, sj, s0, s1, sadd, sz):
    n_nodes, width = agg_sh.shape
    _, _, n_ch, ch = ei_hbm.shape
    rpt = n_nodes // NS

    c = lax.axis_index("c")
    s = lax.axis_index("s")
    wid = s * NC + c

    pltpu.async_copy(ei_hbm.at[0, wid], src_w, sj)
    pltpu.async_copy(ei_hbm.at[1, wid, 0], da, si)

    def _zrow(i, _):
        def _zlane(j, _):
            r1[i, pl.ds(j * 16, 16)] = jnp.zeros((16,), jnp.float32)
            return 0
        return lax.fori_loop(0, width // 16, _zlane, 0)
    lax.fori_loop(0, ch, _zrow, 0)

    zfull = (-(-n_nodes // NS) + ch - 1) // ch
    z0 = s * zfull * ch
    nz = jnp.minimum(zfull, jnp.maximum(0, -(-(n_nodes - z0) // ch)))

    def _zc(i, _):
        pltpu.async_copy(r1, agg_sh.at[pl.ds(z0 + i * ch, ch)], sz)
        return 0
    lax.fori_loop(0, nz, _zc, 0)

    pltpu.async_copy(ei_hbm.at[1, wid, 1], db, si)
    pltpu.make_async_copy(ei_hbm.at[0, wid], src_w, sj).wait()
    pltpu.async_copy(z_hbm.at[src_w.at[0]], r0, s0)

    def _zw(i, _):
        pltpu.make_async_copy(r1, agg_sh.at[pl.ds(z0, ch)], sz).wait()
        return 0
    lax.fori_loop(0, nz, _zw, 0)
    plsc.subcore_barrier()

    def _pair(k, _):
        a = 2 * k
        pltpu.async_copy(z_hbm.at[src_w.at[a + 1]], r1, s1)
        pltpu.make_async_copy(ei_hbm.at[1, wid, 0], da, si).wait()
        pltpu.make_async_copy(ei_hbm.at[1, wid, 0], db, si).wait()
        pltpu.make_async_copy(z_hbm.at[src_w.at[0]], r0, s0).wait()
        pltpu.sync_copy(r0, agg_sh.at[da], add=True)
        pltpu.async_copy(ei_hbm.at[1, wid, a + 2], da, si)
        pltpu.async_copy(z_hbm.at[src_w.at[a + 2]], r0, s0)
        pltpu.make_async_copy(z_hbm.at[src_w.at[0]], r1, s1).wait()
        pltpu.sync_copy(r1, agg_sh.at[db], add=True)

        @pl.when(a + 3 < n_ch)
        def _():
            pltpu.async_copy(ei_hbm.at[1, wid, a + 3], db, si)
        return 0
    lax.fori_loop(0, (n_ch - 1) // 2, _pair, 0)
    pltpu.make_async_copy(ei_hbm.at[1, wid, 0], da, si).wait()
    pltpu.make_async_copy(z_hbm.at[src_w.at[0]], r0, s0).wait()
    pltpu.sync_copy(r0, agg_sh.at[da], add=True)

    plsc.subcore_barrier()

    @pl.when(c == 0)
    def _():
        pltpu.sync_copy(agg_sh.at[pl.ds(s * rpt, rpt)], o0.at[pl.ds(s * rpt, rpt)])

    @pl.when(c == 1)
    def _():
        pltpu.sync_copy(agg_sh.at[pl.ds(s * rpt, rpt)], o1.at[pl.ds(s * rpt, rpt)])


@jax.jit
def _sc_agg(ei, z):
    n_nodes, width = z.shape
    mesh = plsc.VectorSubcoreMesh(core_axis_name="c", subcore_axis_name="s",
                                  num_cores=NC, num_subcores=NS)
    f = pl.kernel(
        _sc_agg_body,
        out_type=[jax.ShapeDtypeStruct((n_nodes, width), jnp.float32)] * 2,
        mesh=mesh,
        scratch_types=[
            pltpu.VMEM((ei.shape[2], CH), jnp.int32),
            pltpu.VMEM((CH,), jnp.int32),
            pltpu.VMEM((CH,), jnp.int32),
            pltpu.VMEM((CH, width), jnp.float32),
            pltpu.VMEM((CH, width), jnp.float32),
            pltpu.VMEM_SHARED((n_nodes, width), jnp.float32),
            pltpu.SemaphoreType.DMA,
            pltpu.SemaphoreType.DMA,
            pltpu.SemaphoreType.DMA,
            pltpu.SemaphoreType.DMA,
            pltpu.SemaphoreType.DMA,
            pltpu.SemaphoreType.DMA,
        ],
        compiler_params=pltpu.CompilerParams(use_tc_tiling_on_sc=False),
    )
    return f(ei, z)


def _proj0_body(x_ref, w_ref, o_ref):
    bn = x_ref.shape[0]
    o_ref[:, :128] = jnp.dot(x_ref[...], w_ref[...],
                             preferred_element_type=jnp.float32)
    col = lax.broadcasted_iota(jnp.int32, (bn, 16), 1)
    o_ref[:, 128:144] = jnp.where(col == 0, 1.0, 0.0)


def _mid_body(a0_ref, a1_ref, z_ref, b_ref, w_ref, o_ref, inv_ref):
    a0 = a0_ref[...]
    a1 = a1_ref[...]
    z = z_ref[...]
    d = a0[:, 128:129] + a1[:, 128:129] + 1.0
    inv = 1.0 / d
    h = (a0[:, :128] + a1[:, :128] + z[:, :128]) * inv + b_ref[...][None, :]
    h = jnp.maximum(h, 0.0)
    o_ref[...] = jnp.dot(h, w_ref[...], preferred_element_type=jnp.float32)
    inv_ref[...] = inv


def _mid2_body(a0_ref, a1_ref, z_ref, inv_ref, b_ref, w_ref, o_ref):
    bn = z_ref.shape[0]
    ncp = o_ref.shape[1]
    ncls = w_ref.shape[1]
    inv = inv_ref[...]
    h = (a0_ref[...] + a1_ref[...] + z_ref[...]) * inv + b_ref[...][None, :]
    h = jnp.maximum(h, 0.0)
    mm = jnp.dot(h, w_ref[...], preferred_element_type=jnp.float32)
    o_ref[...] = jnp.concatenate(
        [mm, jnp.zeros((bn, ncp - ncls), jnp.float32)], axis=1)


def _final_body(a0_ref, a1_ref, z_ref, inv_ref, b_ref, o_ref):
    bn, w = z_ref.shape
    ncls = o_ref.shape[1]
    bp = jnp.concatenate(
        [b_ref[...], jnp.zeros((w - ncls,), jnp.float32)])[None, :]
    t = (a0_ref[...] + a1_ref[...] + z_ref[...]) * inv_ref[...] + bp
    col = lax.broadcasted_iota(jnp.int32, (bn, w), 1)
    valid = col < ncls
    tm = jnp.where(valid, t, -1e30)
    m = jnp.max(tm, axis=1, keepdims=True)
    e = jnp.exp(tm - m)
    ssum = jnp.sum(e, axis=1, keepdims=True)
    o_ref[...] = (t - m - jnp.log(ssum))[:, :ncls]


def _rows_spec(bn, w):
    return pl.BlockSpec((bn, w), lambda i: (i, 0))


def _full_spec(shape):
    return pl.BlockSpec(shape, lambda i: tuple(0 for _ in shape))


def kernel(x, edge_index, W0, b0, W1, b1, W2, b2):
    n, d = x.shape
    dh = W1.shape[0]
    ncls = W2.shape[1]
    e = edge_index.shape[1]
    epw = e // NW
    ei = edge_index.astype(jnp.int32).reshape(2, NW, epw // CH, CH)

    bn = 2000
    grid = (n // bn,)

    z1 = pl.pallas_call(
        _proj0_body,
        grid=grid,
        in_specs=[_rows_spec(bn, d), _full_spec((d, dh))],
        out_specs=_rows_spec(bn, dh + 16),
        out_shape=jax.ShapeDtypeStruct((n, dh + 16), jnp.float32),
    )(x, W0)

    a0, a1 = _sc_agg(ei, z1)

    z2, inv = pl.pallas_call(
        _mid_body,
        grid=grid,
        in_specs=[_rows_spec(bn, dh + 16), _rows_spec(bn, dh + 16),
                  _rows_spec(bn, dh + 16), _full_spec((dh,)),
                  _full_spec((dh, dh))],
        out_specs=[_rows_spec(bn, dh), pl.BlockSpec((bn, 1), lambda i: (i, 0))],
        out_shape=[jax.ShapeDtypeStruct((n, dh), jnp.float32),
                   jax.ShapeDtypeStruct((n, 1), jnp.float32)],
    )(a0, a1, z1, b0, W1)

    a0, a1 = _sc_agg(ei, z2)

    ncp = 48
    z3 = pl.pallas_call(
        _mid2_body,
        grid=grid,
        in_specs=[_rows_spec(bn, dh), _rows_spec(bn, dh), _rows_spec(bn, dh),
                  pl.BlockSpec((bn, 1), lambda i: (i, 0)),
                  _full_spec((dh,)), _full_spec((dh, ncls))],
        out_specs=_rows_spec(bn, ncp),
        out_shape=jax.ShapeDtypeStruct((n, ncp), jnp.float32),
    )(a0, a1, z2, inv, b1, W2)

    a0, a1 = _sc_agg(ei, z3)

    out = pl.pallas_call(
        _final_body,
        grid=grid,
        in_specs=[_rows_spec(bn, ncp), _rows_spec(bn, ncp), _rows_spec(bn, ncp),
                  pl.BlockSpec((bn, 1), lambda i: (i, 0)),
                  _full_spec((ncls,))],
        out_specs=_rows_spec(bn, ncls),
        out_shape=jax.ShapeDtypeStruct((n, ncls), jnp.float32),
    )(a0, a1, z3, inv, b2)

    return out


# speedup vs baseline: 1.3109x; 1.0031x over previous
import functools

import jax
import jax.numpy as jnp
from jax import lax
from jax.experimental import pallas as pl
from jax.experimental.pallas import tpu as pltpu
from jax.experimental.pallas import tpu_sc as plsc

NC = 2
NS = 16
NW = NC * NS
CH = 80


def _sc_agg_body(ei_hbm, z_hbm, o0, o1,
                 src_w, da, db, r0, r1, agg_sh, si, sj, s0, s1, sadd, sz):
    n_nodes, width = agg_sh.shape
    _, _, n_ch, ch = ei_hbm.shape
    rpt = n_nodes // NS

    c = lax.axis_index("c")
    s = lax.axis_index("s")
    wid = s * NC + c

    pltpu.async_copy(ei_hbm.at[0, wid], src_w, sj)
    pltpu.async_copy(ei_hbm.at[1, wid, 0], da, si)

    def _zrow(i, _):
        def _zlane(j, _):
            r1[i, pl.ds(j * 16, 16)] = jnp.zeros((16,), jnp.float32)
            return 0
        return lax.fori_loop(0, width // 16, _zlane, 0)
    lax.fori_loop(0, ch, _zrow, 0)

    zfull = (-(-n_nodes // NS) + ch - 1) // ch
    z0 = s * zfull * ch
    nz = jnp.minimum(zfull, jnp.maximum(0, -(-(n_nodes - z0) // ch)))

    def _zc(i, _):
        pltpu.async_copy(r1, agg_sh.at[pl.ds(z0 + i * ch, ch)], sz)
        return 0
    lax.fori_loop(0, nz, _zc, 0)

    pltpu.async_copy(ei_hbm.at[1, wid, 1], db, si)
    pltpu.make_async_copy(ei_hbm.at[0, wid], src_w, sj).wait()
    pltpu.async_copy(z_hbm.at[src_w.at[0]], r0, s0)

    def _zw(i, _):
        pltpu.make_async_copy(r1, agg_sh.at[pl.ds(z0, ch)], sz).wait()
        return 0
    lax.fori_loop(0, nz, _zw, 0)
    plsc.subcore_barrier()

    def _pair(k, _):
        a = 2 * k
        pltpu.async_copy(z_hbm.at[src_w.at[a + 1]], r1, s1)
        pltpu.make_async_copy(ei_hbm.at[1, wid, 0], da, si).wait()
        pltpu.make_async_copy(ei_hbm.at[1, wid, 0], db, si).wait()
        pltpu.make_async_copy(z_hbm.at[src_w.at[0]], r0, s0).wait()
        pltpu.sync_copy(r0, agg_sh.at[da], add=True)
        pltpu.async_copy(ei_hbm.at[1, wid, a + 2], da, si)
        pltpu.async_copy(z_hbm.at[src_w.at[a + 2]], r0, s0)
        pltpu.make_async_copy(z_hbm.at[src_w.at[0]], r1, s1).wait()
        pltpu.sync_copy(r1, agg_sh.at[db], add=True)

        @pl.when(a + 3 < n_ch)
        def _():
            pltpu.async_copy(ei_hbm.at[1, wid, a + 3], db, si)
        return 0
    lax.fori_loop(0, (n_ch - 1) // 2, _pair, 0)
    pltpu.make_async_copy(ei_hbm.at[1, wid, 0], da, si).wait()
    pltpu.make_async_copy(z_hbm.at[src_w.at[0]], r0, s0).wait()
    pltpu.sync_copy(r0, agg_sh.at[da], add=True)

    plsc.subcore_barrier()

    @pl.when(c == 0)
    def _():
        pltpu.sync_copy(agg_sh.at[pl.ds(s * rpt, rpt)], o0.at[pl.ds(s * rpt, rpt)])

    @pl.when(c == 1)
    def _():
        pltpu.sync_copy(agg_sh.at[pl.ds(s * rpt, rpt)], o1.at[pl.ds(s * rpt, rpt)])


def _sc_agg_body2(ei_hbm, z_hbm, o0, o1,
                  src_w, dst_w, r0, r1, agg_sh, sj, s0, s1, sz):
    n_nodes, width = agg_sh.shape
    _, _, n_ch, ch = ei_hbm.shape
    rpt = n_nodes // NS

    c = lax.axis_index("c")
    s = lax.axis_index("s")
    wid = s * NC + c

    pltpu.async_copy(ei_hbm.at[0, wid], src_w, sj)
    pltpu.async_copy(ei_hbm.at[1, wid], dst_w, sj)

    def _zrow(i, _):
        def _zlane(j, _):
            r1[i, pl.ds(j * 16, 16)] = jnp.zeros((16,), jnp.float32)
            return 0
        return lax.fori_loop(0, width // 16, _zlane, 0)
    lax.fori_loop(0, ch, _zrow, 0)

    zfull = (-(-n_nodes // NS) + ch - 1) // ch
    z0 = s * zfull * ch
    nz = jnp.minimum(zfull, jnp.maximum(0, -(-(n_nodes - z0) // ch)))

    def _zc(i, _):
        pltpu.async_copy(r1, agg_sh.at[pl.ds(z0 + i * ch, ch)], sz)
        return 0
    lax.fori_loop(0, nz, _zc, 0)

    pltpu.make_async_copy(ei_hbm.at[0, wid], src_w, sj).wait()
    pltpu.make_async_copy(ei_hbm.at[0, wid], dst_w, sj).wait()
    pltpu.async_copy(z_hbm.at[src_w.at[0]], r0, s0)

    def _zw(i, _):
        pltpu.make_async_copy(r1, agg_sh.at[pl.ds(z0, ch)], sz).wait()
        return 0
    lax.fori_loop(0, nz, _zw, 0)
    plsc.subcore_barrier()

    def _pair(k, _):
        a = 2 * k
        pltpu.async_copy(z_hbm.at[src_w.at[a + 1]], r1, s1)
        pltpu.make_async_copy(z_hbm.at[src_w.at[0]], r0, s0).wait()
        pltpu.sync_copy(r0, agg_sh.at[dst_w.at[a]], add=True)
        pltpu.async_copy(z_hbm.at[src_w.at[a + 2]], r0, s0)
        pltpu.make_async_copy(z_hbm.at[src_w.at[0]], r1, s1).wait()
        pltpu.sync_copy(r1, agg_sh.at[dst_w.at[a + 1]], add=True)
        return 0
    lax.fori_loop(0, (n_ch - 1) // 2, _pair, 0)
    pltpu.make_async_copy(z_hbm.at[src_w.at[0]], r0, s0).wait()
    pltpu.sync_copy(r0, agg_sh.at[dst_w.at[n_ch - 1]], add=True)

    plsc.subcore_barrier()

    @pl.when(c == 0)
    def _():
        pltpu.sync_copy(agg_sh.at[pl.ds(s * rpt, rpt)], o0.at[pl.ds(s * rpt, rpt)])

    @pl.when(c == 1)
    def _():
        pltpu.sync_copy(agg_sh.at[pl.ds(s * rpt, rpt)], o1.at[pl.ds(s * rpt, rpt)])


_SPMEM_BUDGET = 2_040_000


@jax.jit
def _sc_agg(ei, z):
    n_nodes, width = z.shape
    n_ch = ei.shape[2]
    mesh = plsc.VectorSubcoreMesh(core_axis_name="c", subcore_axis_name="s",
                                  num_cores=NC, num_subcores=NS)
    slab_words = 16 * (2 * n_ch * CH + 2 * CH * width) + n_nodes * width
    if slab_words <= _SPMEM_BUDGET:
        body = _sc_agg_body2
        scratch = [
            pltpu.VMEM((n_ch, CH), jnp.int32),
            pltpu.VMEM((n_ch, CH), jnp.int32),
            pltpu.VMEM((CH, width), jnp.float32),
            pltpu.VMEM((CH, width), jnp.float32),
            pltpu.VMEM_SHARED((n_nodes, width), jnp.float32),
            pltpu.SemaphoreType.DMA,
            pltpu.SemaphoreType.DMA,
            pltpu.SemaphoreType.DMA,
            pltpu.SemaphoreType.DMA,
        ]
    else:
        body = _sc_agg_body
        scratch = [
            pltpu.VMEM((n_ch, CH), jnp.int32),
            pltpu.VMEM((CH,), jnp.int32),
            pltpu.VMEM((CH,), jnp.int32),
            pltpu.VMEM((CH, width), jnp.float32),
            pltpu.VMEM((CH, width), jnp.float32),
            pltpu.VMEM_SHARED((n_nodes, width), jnp.float32),
            pltpu.SemaphoreType.DMA,
            pltpu.SemaphoreType.DMA,
            pltpu.SemaphoreType.DMA,
            pltpu.SemaphoreType.DMA,
            pltpu.SemaphoreType.DMA,
            pltpu.SemaphoreType.DMA,
        ]
    f = pl.kernel(
        body,
        out_type=[jax.ShapeDtypeStruct((n_nodes, width), jnp.float32)] * 2,
        mesh=mesh,
        scratch_types=scratch,
        compiler_params=pltpu.CompilerParams(use_tc_tiling_on_sc=False),
    )
    return f(ei, z)


def _proj0_body(x_ref, w_ref, o_ref):
    bn = x_ref.shape[0]
    o_ref[:, :128] = jnp.dot(x_ref[...], w_ref[...],
                             preferred_element_type=jnp.float32)
    col = lax.broadcasted_iota(jnp.int32, (bn, 16), 1)
    o_ref[:, 128:144] = jnp.where(col == 0, 1.0, 0.0)


def _mid_body(a0_ref, a1_ref, z_ref, b_ref, w_ref, o_ref, inv_ref):
    a0 = a0_ref[...]
    a1 = a1_ref[...]
    z = z_ref[...]
    d = a0[:, 128:129] + a1[:, 128:129] + 1.0
    inv = 1.0 / d
    h = (a0[:, :128] + a1[:, :128] + z[:, :128]) * inv + b_ref[...][None, :]
    h = jnp.maximum(h, 0.0)
    o_ref[...] = jnp.dot(h, w_ref[...], preferred_element_type=jnp.float32)
    inv_ref[...] = inv


def _mid2_body(a0_ref, a1_ref, z_ref, inv_ref, b_ref, w_ref, o_ref):
    bn = z_ref.shape[0]
    ncp = o_ref.shape[1]
    ncls = w_ref.shape[1]
    inv = inv_ref[...]
    h = (a0_ref[...] + a1_ref[...] + z_ref[...]) * inv + b_ref[...][None, :]
    h = jnp.maximum(h, 0.0)
    mm = jnp.dot(h, w_ref[...], preferred_element_type=jnp.float32)
    o_ref[...] = jnp.concatenate(
        [mm, jnp.zeros((bn, ncp - ncls), jnp.float32)], axis=1)


def _final_body(a0_ref, a1_ref, z_ref, inv_ref, b_ref, o_ref):
    bn, w = z_ref.shape
    ncls = o_ref.shape[1]
    bp = jnp.concatenate(
        [b_ref[...], jnp.zeros((w - ncls,), jnp.float32)])[None, :]
    t = (a0_ref[...] + a1_ref[...] + z_ref[...]) * inv_ref[...] + bp
    col = lax.broadcasted_iota(jnp.int32, (bn, w), 1)
    valid = col < ncls
    tm = jnp.where(valid, t, -1e30)
    m = jnp.max(tm, axis=1, keepdims=True)
    e = jnp.exp(tm - m)
    ssum = jnp.sum(e, axis=1, keepdims=True)
    o_ref[...] = (t - m - jnp.log(ssum))[:, :ncls]


def _rows_spec(bn, w):
    return pl.BlockSpec((bn, w), lambda i: (i, 0))


def _full_spec(shape):
    return pl.BlockSpec(shape, lambda i: tuple(0 for _ in shape))


def kernel(x, edge_index, W0, b0, W1, b1, W2, b2):
    n, d = x.shape
    dh = W1.shape[0]
    ncls = W2.shape[1]
    e = edge_index.shape[1]
    epw = e // NW
    ei = edge_index.astype(jnp.int32).reshape(2, NW, epw // CH, CH)

    bn = 2000
    grid = (n // bn,)

    z1 = pl.pallas_call(
        _proj0_body,
        grid=grid,
        in_specs=[_rows_spec(bn, d), _full_spec((d, dh))],
        out_specs=_rows_spec(bn, dh + 16),
        out_shape=jax.ShapeDtypeStruct((n, dh + 16), jnp.float32),
    )(x, W0)

    a0, a1 = _sc_agg(ei, z1)

    z2, inv = pl.pallas_call(
        _mid_body,
        grid=grid,
        in_specs=[_rows_spec(bn, dh + 16), _rows_spec(bn, dh + 16),
                  _rows_spec(bn, dh + 16), _full_spec((dh,)),
                  _full_spec((dh, dh))],
        out_specs=[_rows_spec(bn, dh), pl.BlockSpec((bn, 1), lambda i: (i, 0))],
        out_shape=[jax.ShapeDtypeStruct((n, dh), jnp.float32),
                   jax.ShapeDtypeStruct((n, 1), jnp.float32)],
    )(a0, a1, z1, b0, W1)

    a0, a1 = _sc_agg(ei, z2)

    ncp = 48
    z3 = pl.pallas_call(
        _mid2_body,
        grid=grid,
        in_specs=[_rows_spec(bn, dh), _rows_spec(bn, dh), _rows_spec(bn, dh),
                  pl.BlockSpec((bn, 1), lambda i: (i, 0)),
                  _full_spec((dh,)), _full_spec((dh, ncls))],
        out_specs=_rows_spec(bn, ncp),
        out_shape=jax.ShapeDtypeStruct((n, ncp), jnp.float32),
    )(a0, a1, z2, inv, b1, W2)

    a0, a1 = _sc_agg(ei, z3)

    out = pl.pallas_call(
        _final_body,
        grid=grid,
        in_specs=[_rows_spec(bn, ncp), _rows_spec(bn, ncp), _rows_spec(bn, ncp),
                  pl.BlockSpec((bn, 1), lambda i: (i, 0)),
                  _full_spec((ncls,))],
        out_specs=_rows_spec(bn, ncls),
        out_shape=jax.ShapeDtypeStruct((n, ncls), jnp.float32),
    )(a0, a1, z3, inv, b2)

    return out
